# baseline (speedup 1.0000x reference)
import functools

import jax
import jax.numpy as jnp
from jax import lax
from jax.experimental import pallas as pl
from jax.experimental.pallas import tpu as pltpu
from jax.experimental.pallas import tpu_sc as plsc

_LANES = 128
_NC = 2
_NS = 16
_KB = 80
_ZC = 40


def _segment_messages(h2, ep2, src, dst, n_pad, e, p):
    ppc = p // _NC
    ept = e // _NS
    nb = ept // _KB
    assert nb % 2 == 1, "pipeline assumes an odd batch count"
    npt = n_pad // _NS
    nz = npt // _ZC

    mesh = plsc.VectorSubcoreMesh(
        core_axis_name="c", subcore_axis_name="s",
        num_cores=_NC, num_subcores=_NS)

    @functools.partial(
        pl.kernel,
        out_type=jax.ShapeDtypeStruct((p * n_pad, _LANES), jnp.float32),
        mesh=mesh,
        scratch_types=[
            pltpu.VMEM((_KB,), jnp.int32),
            pltpu.VMEM((_KB,), jnp.int32),
            pltpu.VMEM((_KB,), jnp.int32),
            pltpu.VMEM((_KB,), jnp.int32),
            pltpu.VMEM((_KB,), jnp.int32),
            pltpu.VMEM((_KB,), jnp.int32),
            pltpu.VMEM((_KB, _LANES), jnp.float32),
            pltpu.VMEM((_KB, _LANES), jnp.float32),
            pltpu.VMEM((_KB, _LANES), jnp.float32),
            pltpu.VMEM((_KB, _LANES), jnp.float32),
            pltpu.VMEM((_ZC, _LANES), jnp.float32),
            pltpu.VMEM_SHARED((n_pad, _LANES), jnp.float32),
            pltpu.SemaphoreType.DMA,
            pltpu.SemaphoreType.DMA,
            pltpu.SemaphoreType.DMA,
            pltpu.SemaphoreType.DMA,
        ],
    )
    def aggr_kernel(h_hbm, ep_hbm, src_hbm, dst_hbm, out_hbm,
                    src0, src1, gid0, gid1, dst0, dst1,
                    rows0, rows1, epv0, epv1, zb_v, acc,
                    si0, si1, sg0, sg1):
        c = lax.axis_index("c")
        s = lax.axis_index("s")

        def zero_zb(r, _):
            for j in range(_LANES // 16):
                zb_v[r, pl.ds(j * 16, 16)] = jnp.zeros((16,), jnp.float32)
            return 0

        lax.fori_loop(0, _ZC, zero_zb, 0)

        def idx_start(b, sv, dv, sem):
            base = s * ept + b * _KB
            pltpu.async_copy(src_hbm.at[pl.ds(base, _KB)], sv, sem)
            pltpu.async_copy(dst_hbm.at[pl.ds(base, _KB)], dv, sem)

        def idx_wait(sv, dv, sem):
            pltpu.make_async_copy(src_hbm.at[pl.ds(0, _KB)], sv, sem).wait()
            pltpu.make_async_copy(dst_hbm.at[pl.ds(0, _KB)], dv, sem).wait()

        def gid_compute(sv, gv, q):
            def gid(i, _):
                sl = pl.ds(i * 16, 16)
                gv[sl] = sv[sl] * p + q
                return 0
            lax.fori_loop(0, _KB // 16, gid, 0)

        def gather_start(b, gv, rv, ev, sem, q):
            base = s * ept + b * _KB
            pltpu.async_copy(h_hbm.at[gv], rv, sem)
            pltpu.async_copy(ep_hbm.at[pl.ds(q * e + base, _KB)], ev, sem)

        def gather_wait(gv, rv, ev, sem):
            pltpu.make_async_copy(h_hbm.at[gv], rv, sem).wait()
            pltpu.make_async_copy(ep_hbm.at[pl.ds(0, _KB)], ev, sem).wait()

        def compute_scatter(rv, ev, dv):
            @plsc.parallel_loop(0, _KB, 1, unroll=2)
            def relu_row(r):
                for j in range(_LANES // 16):
                    sl = pl.ds(j * 16, 16)
                    rv[r, sl] = jnp.maximum(rv[r, sl] + ev[r, sl], 0.0)
            pltpu.sync_copy(rv, acc.at[dv], add=True)

        for ql in range(ppc):
            q = c * ppc + ql

            def zero_acc(z, _):
                pltpu.sync_copy(zb_v, acc.at[pl.ds(s * npt + z * _ZC, _ZC)])
                return 0

            lax.fori_loop(0, nz, zero_acc, 0)
            plsc.subcore_barrier()

            idx_start(0, src0, dst0, si0)
            idx_wait(src0, dst0, si0)
            gid_compute(src0, gid0, q)
            gather_start(0, gid0, rows0, epv0, sg0, q)
            idx_start(1, src1, dst1, si1)

            def pair(t, _):
                b = 2 * t
                idx_wait(src1, dst1, si1)
                gid_compute(src1, gid1, q)
                gather_start(b + 1, gid1, rows1, epv1, sg1, q)
                gather_wait(gid0, rows0, epv0, sg0)
                compute_scatter(rows0, epv0, dst0)
                idx_start(b + 2, src0, dst0, si0)
                idx_wait(src0, dst0, si0)
                gid_compute(src0, gid0, q)
                gather_start(b + 2, gid0, rows0, epv0, sg0, q)
                gather_wait(gid1, rows1, epv1, sg1)
                compute_scatter(rows1, epv1, dst1)

                @pl.when(b + 3 < nb)
                def _():
                    idx_start(b + 3, src1, dst1, si1)
                return 0

            lax.fori_loop(0, (nb - 1) // 2, pair, 0)
            gather_wait(gid0, rows0, epv0, sg0)
            compute_scatter(rows0, epv0, dst0)

            plsc.subcore_barrier()

            off = s * npt
            pltpu.sync_copy(
                acc.at[pl.ds(off, npt)],
                out_hbm.at[pl.ds(q * n_pad + off, npt)])

    return aggr_kernel(h2, ep2, src, dst)


def _edge_proj(ef, W, b, p):
    e, de = ef.shape
    dout = W.shape[1]
    be_blk = 4000

    def kern(ef_ref, w_ref, b_ref, out_ref):
        acc = jnp.dot(ef_ref[...], w_ref[...],
                      preferred_element_type=jnp.float32) + b_ref[...]
        for qq in range(p):
            out_ref[qq] = acc[:, qq * _LANES:(qq + 1) * _LANES]

    return pl.pallas_call(
        kern,
        grid=(e // be_blk,),
        in_specs=[
            pl.BlockSpec((be_blk, de), lambda i: (i, 0)),
            pl.BlockSpec((de, dout), lambda i: (0, 0)),
            pl.BlockSpec((1, dout), lambda i: (0, 0)),
        ],
        out_specs=pl.BlockSpec((p, be_blk, _LANES), lambda i: (0, i, 0)),
        out_shape=jax.ShapeDtypeStruct((p, e, _LANES), jnp.float32),
    )(ef, W, b.reshape(1, dout))


def _node_mlp1(x, aggr, Wa, ba, Wb, bb):
    n, din = x.shape
    p = aggr.shape[0]
    dh = Wa.shape[1]
    bn = 2000

    def kern(x_ref, a_ref, wa_ref, ba_ref, wb_ref, bb_ref, out_ref):
        g = x_ref[...] + jnp.concatenate(
            [a_ref[qq] for qq in range(p)], axis=1)
        t = jnp.maximum(
            jnp.dot(g, wa_ref[...], preferred_element_type=jnp.float32)
            + ba_ref[...], 0.0)
        out_ref[...] = jnp.tanh(
            jnp.dot(t, wb_ref[...], preferred_element_type=jnp.float32)
            + bb_ref[...])

    return pl.pallas_call(
        kern,
        grid=(n // bn,),
        in_specs=[
            pl.BlockSpec((bn, din), lambda i: (i, 0)),
            pl.BlockSpec((p, bn, _LANES), lambda i: (0, i, 0)),
            pl.BlockSpec((din, dh), lambda i: (0, 0)),
            pl.BlockSpec((1, dh), lambda i: (0, 0)),
            pl.BlockSpec((dh, dh), lambda i: (0, 0)),
            pl.BlockSpec((1, dh), lambda i: (0, 0)),
        ],
        out_specs=pl.BlockSpec((bn, dh), lambda i: (i, 0)),
        out_shape=jax.ShapeDtypeStruct((n, dh), jnp.float32),
    )(x, aggr, Wa, ba.reshape(1, dh), Wb, bb.reshape(1, dh))


def _node_mlp2_head(h, aggr, Wa, ba, Wb, bb, Wf1, bf1, Wf2, bf2):
    n, dh = h.shape
    p = aggr.shape[0]
    dout = Wf2.shape[1]
    bn = 2000

    def kern(h_ref, a_ref, wa_ref, ba_ref, wb_ref, bb_ref,
             wf1_ref, bf1_ref, wf2_ref, bf2_ref, out_ref):
        g = h_ref[...] + jnp.concatenate(
            [a_ref[qq] for qq in range(p)], axis=1)
        t = jnp.maximum(
            jnp.dot(g, wa_ref[...], preferred_element_type=jnp.float32)
            + ba_ref[...], 0.0)
        t = jnp.tanh(
            jnp.dot(t, wb_ref[...], preferred_element_type=jnp.float32)
            + bb_ref[...])
        t = jnp.tanh(
            jnp.dot(t, wf1_ref[...], preferred_element_type=jnp.float32)
            + bf1_ref[...])
        out_ref[...] = (
            jnp.dot(t, wf2_ref[...], preferred_element_type=jnp.float32)
            + bf2_ref[...])

    return pl.pallas_call(
        kern,
        grid=(n // bn,),
        in_specs=[
            pl.BlockSpec((bn, dh), lambda i: (i, 0)),
            pl.BlockSpec((p, bn, _LANES), lambda i: (0, i, 0)),
            pl.BlockSpec((dh, dh), lambda i: (0, 0)),
            pl.BlockSpec((1, dh), lambda i: (0, 0)),
            pl.BlockSpec((dh, dh), lambda i: (0, 0)),
            pl.BlockSpec((1, dh), lambda i: (0, 0)),
            pl.BlockSpec((dh, dh), lambda i: (0, 0)),
            pl.BlockSpec((1, dh), lambda i: (0, 0)),
            pl.BlockSpec((dh, dout), lambda i: (0, 0)),
            pl.BlockSpec((1, dout), lambda i: (0, 0)),
        ],
        out_specs=pl.BlockSpec((bn, dout), lambda i: (i, 0)),
        out_shape=jax.ShapeDtypeStruct((n, dout), jnp.float32),
    )(h, aggr, Wa, ba.reshape(1, dh), Wb, bb.reshape(1, dh),
      Wf1, bf1.reshape(1, dh), Wf2, bf2.reshape(1, dout))


def kernel(x, edge_index, edge_feats, We1, be1, W1a, b1a, W1b, b1b,
           We2, be2, W2a, b2a, W2b, b2b, Wf1, bf1, Wf2, bf2):
    n, din = x.shape
    e = edge_index.shape[1]
    dh = W1a.shape[1]
    src = edge_index[0]
    dst = edge_index[1]
    p1 = din // _LANES
    p2 = dh // _LANES
    n_pad = -(-n // (_NS * _ZC)) * (_NS * _ZC)

    ep1 = _edge_proj(edge_feats, We1, be1, p1)
    aggr1 = _segment_messages(
        x.reshape(n * p1, _LANES), ep1.reshape(p1 * e, _LANES),
        src, dst, n_pad, e, p1).reshape(p1, n_pad, _LANES)
    h = _node_mlp1(x, aggr1, W1a, b1a, W1b, b1b)

    ep2 = _edge_proj(edge_feats, We2, be2, p2)
    aggr2 = _segment_messages(
        h.reshape(n * p2, _LANES), ep2.reshape(p2 * e, _LANES),
        src, dst, n_pad, e, p2).reshape(p2, n_pad, _LANES)
    out = _node_mlp2_head(h, aggr2, W2a, b2a, W2b, b2b, Wf1, bf1, Wf2, bf2)
    return out


# speedup vs baseline: 1.0022x; 1.0022x over previous
import functools

import jax
import jax.numpy as jnp
from jax import lax
from jax.experimental import pallas as pl
from jax.experimental.pallas import tpu as pltpu
from jax.experimental.pallas import tpu_sc as plsc

_LANES = 128
_NC = 2
_NS = 16
_KB = 80
_ZC = 40


def _segment_messages(h2, ep2, src, dst, n_pad, e, p):
    ppc = p // _NC
    ept = e // _NS
    nb = ept // _KB
    assert nb % 2 == 1, "pipeline assumes an odd batch count"
    npt = n_pad // _NS
    nz = npt // _ZC

    mesh = plsc.VectorSubcoreMesh(
        core_axis_name="c", subcore_axis_name="s",
        num_cores=_NC, num_subcores=_NS)

    @functools.partial(
        pl.kernel,
        out_type=jax.ShapeDtypeStruct((p * n_pad, _LANES), jnp.float32),
        mesh=mesh,
        scratch_types=[
            pltpu.VMEM((_KB,), jnp.int32),
            pltpu.VMEM((_KB,), jnp.int32),
            pltpu.VMEM((_KB,), jnp.int32),
            pltpu.VMEM((_KB,), jnp.int32),
            pltpu.VMEM((_KB,), jnp.int32),
            pltpu.VMEM((_KB,), jnp.int32),
            pltpu.VMEM((_KB, _LANES), jnp.float32),
            pltpu.VMEM((_KB, _LANES), jnp.float32),
            pltpu.VMEM((_KB, _LANES), jnp.float32),
            pltpu.VMEM((_KB, _LANES), jnp.float32),
            pltpu.VMEM_SHARED((n_pad, _LANES), jnp.float32),
            pltpu.SemaphoreType.DMA,
            pltpu.SemaphoreType.DMA,
            pltpu.SemaphoreType.DMA,
            pltpu.SemaphoreType.DMA,
        ],
    )
    def aggr_kernel(h_hbm, ep_hbm, src_hbm, dst_hbm, z_hbm, out_hbm,
                    src0, src1, gid0, gid1, dst0, dst1,
                    rows0, rows1, epv0, epv1, acc,
                    si0, si1, sg0, sg1):
        c = lax.axis_index("c")
        s = lax.axis_index("s")

        def idx_start(b, sv, dv, sem):
            base = s * ept + b * _KB
            pltpu.async_copy(src_hbm.at[pl.ds(base, _KB)], sv, sem)
            pltpu.async_copy(dst_hbm.at[pl.ds(base, _KB)], dv, sem)

        def idx_wait(sv, dv, sem):
            pltpu.make_async_copy(src_hbm.at[pl.ds(0, _KB)], sv, sem).wait()
            pltpu.make_async_copy(dst_hbm.at[pl.ds(0, _KB)], dv, sem).wait()

        def gid_compute(sv, gv, q):
            def gid(i, _):
                sl = pl.ds(i * 16, 16)
                gv[sl] = sv[sl] * p + q
                return 0
            lax.fori_loop(0, _KB // 16, gid, 0)

        def gather_start(b, gv, rv, ev, sem, q):
            base = s * ept + b * _KB
            pltpu.async_copy(h_hbm.at[gv], rv, sem)
            pltpu.async_copy(ep_hbm.at[pl.ds(q * e + base, _KB)], ev, sem)

        def gather_wait(gv, rv, ev, sem):
            pltpu.make_async_copy(h_hbm.at[gv], rv, sem).wait()
            pltpu.make_async_copy(ep_hbm.at[pl.ds(0, _KB)], ev, sem).wait()

        def compute_scatter(rv, ev, dv):
            def relu_row(r, _):
                for j in range(_LANES // 16):
                    sl = pl.ds(j * 16, 16)
                    rv[r, sl] = jnp.maximum(rv[r, sl] + ev[r, sl], 0.0)
                return 0
            lax.fori_loop(0, _KB, relu_row, 0)
            pltpu.sync_copy(rv, acc.at[dv], add=True)

        for ql in range(ppc):
            q = c * ppc + ql

            pltpu.sync_copy(z_hbm.at[pl.ds(s * npt, npt)],
                            acc.at[pl.ds(s * npt, npt)])
            plsc.subcore_barrier()

            idx_start(0, src0, dst0, si0)
            idx_wait(src0, dst0, si0)
            gid_compute(src0, gid0, q)
            gather_start(0, gid0, rows0, epv0, sg0, q)
            idx_start(1, src1, dst1, si1)

            def pair(t, _):
                b = 2 * t
                idx_wait(src1, dst1, si1)
                gid_compute(src1, gid1, q)
                gather_start(b + 1, gid1, rows1, epv1, sg1, q)
                gather_wait(gid0, rows0, epv0, sg0)
                compute_scatter(rows0, epv0, dst0)
                idx_start(b + 2, src0, dst0, si0)
                idx_wait(src0, dst0, si0)
                gid_compute(src0, gid0, q)
                gather_start(b + 2, gid0, rows0, epv0, sg0, q)
                gather_wait(gid1, rows1, epv1, sg1)
                compute_scatter(rows1, epv1, dst1)

                @pl.when(b + 3 < nb)
                def _():
                    idx_start(b + 3, src1, dst1, si1)
                return 0

            lax.fori_loop(0, (nb - 1) // 2, pair, 0)
            gather_wait(gid0, rows0, epv0, sg0)
            compute_scatter(rows0, epv0, dst0)

            plsc.subcore_barrier()

            off = s * npt
            pltpu.sync_copy(
                acc.at[pl.ds(off, npt)],
                out_hbm.at[pl.ds(q * n_pad + off, npt)])

    return aggr_kernel(h2, ep2, src, dst,
                       jnp.zeros((n_pad, _LANES), jnp.float32))


def _edge_proj(ef, W, b, p):
    e, de = ef.shape
    dout = W.shape[1]
    be_blk = 4000

    def kern(ef_ref, w_ref, b_ref, out_ref):
        acc = jnp.dot(ef_ref[...], w_ref[...],
                      preferred_element_type=jnp.float32) + b_ref[...]
        for qq in range(p):
            out_ref[qq] = acc[:, qq * _LANES:(qq + 1) * _LANES]

    return pl.pallas_call(
        kern,
        grid=(e // be_blk,),
        in_specs=[
            pl.BlockSpec((be_blk, de), lambda i: (i, 0)),
            pl.BlockSpec((de, dout), lambda i: (0, 0)),
            pl.BlockSpec((1, dout), lambda i: (0, 0)),
        ],
        out_specs=pl.BlockSpec((p, be_blk, _LANES), lambda i: (0, i, 0)),
        out_shape=jax.ShapeDtypeStruct((p, e, _LANES), jnp.float32),
    )(ef, W, b.reshape(1, dout))


def _node_mlp1(x, aggr, Wa, ba, Wb, bb):
    n, din = x.shape
    p = aggr.shape[0]
    dh = Wa.shape[1]
    bn = 2000

    def kern(x_ref, a_ref, wa_ref, ba_ref, wb_ref, bb_ref, out_ref):
        g = x_ref[...] + jnp.concatenate(
            [a_ref[qq] for qq in range(p)], axis=1)
        t = jnp.maximum(
            jnp.dot(g, wa_ref[...], preferred_element_type=jnp.float32)
            + ba_ref[...], 0.0)
        out_ref[...] = jnp.tanh(
            jnp.dot(t, wb_ref[...], preferred_element_type=jnp.float32)
            + bb_ref[...])

    return pl.pallas_call(
        kern,
        grid=(n // bn,),
        in_specs=[
            pl.BlockSpec((bn, din), lambda i: (i, 0)),
            pl.BlockSpec((p, bn, _LANES), lambda i: (0, i, 0)),
            pl.BlockSpec((din, dh), lambda i: (0, 0)),
            pl.BlockSpec((1, dh), lambda i: (0, 0)),
            pl.BlockSpec((dh, dh), lambda i: (0, 0)),
            pl.BlockSpec((1, dh), lambda i: (0, 0)),
        ],
        out_specs=pl.BlockSpec((bn, dh), lambda i: (i, 0)),
        out_shape=jax.ShapeDtypeStruct((n, dh), jnp.float32),
    )(x, aggr, Wa, ba.reshape(1, dh), Wb, bb.reshape(1, dh))


def _node_mlp2_head(h, aggr, Wa, ba, Wb, bb, Wf1, bf1, Wf2, bf2):
    n, dh = h.shape
    p = aggr.shape[0]
    dout = Wf2.shape[1]
    bn = 2000

    def kern(h_ref, a_ref, wa_ref, ba_ref, wb_ref, bb_ref,
             wf1_ref, bf1_ref, wf2_ref, bf2_ref, out_ref):
        g = h_ref[...] + jnp.concatenate(
            [a_ref[qq] for qq in range(p)], axis=1)
        t = jnp.maximum(
            jnp.dot(g, wa_ref[...], preferred_element_type=jnp.float32)
            + ba_ref[...], 0.0)
        t = jnp.tanh(
            jnp.dot(t, wb_ref[...], preferred_element_type=jnp.float32)
            + bb_ref[...])
        t = jnp.tanh(
            jnp.dot(t, wf1_ref[...], preferred_element_type=jnp.float32)
            + bf1_ref[...])
        out_ref[...] = (
            jnp.dot(t, wf2_ref[...], preferred_element_type=jnp.float32)
            + bf2_ref[...])

    return pl.pallas_call(
        kern,
        grid=(n // bn,),
        in_specs=[
            pl.BlockSpec((bn, dh), lambda i: (i, 0)),
            pl.BlockSpec((p, bn, _LANES), lambda i: (0, i, 0)),
            pl.BlockSpec((dh, dh), lambda i: (0, 0)),
            pl.BlockSpec((1, dh), lambda i: (0, 0)),
            pl.BlockSpec((dh, dh), lambda i: (0, 0)),
            pl.BlockSpec((1, dh), lambda i: (0, 0)),
            pl.BlockSpec((dh, dh), lambda i: (0, 0)),
            pl.BlockSpec((1, dh), lambda i: (0, 0)),
            pl.BlockSpec((dh, dout), lambda i: (0, 0)),
            pl.BlockSpec((1, dout), lambda i: (0, 0)),
        ],
        out_specs=pl.BlockSpec((bn, dout), lambda i: (i, 0)),
        out_shape=jax.ShapeDtypeStruct((n, dout), jnp.float32),
    )(h, aggr, Wa, ba.reshape(1, dh), Wb, bb.reshape(1, dh),
      Wf1, bf1.reshape(1, dh), Wf2, bf2.reshape(1, dout))


def kernel(x, edge_index, edge_feats, We1, be1, W1a, b1a, W1b, b1b,
           We2, be2, W2a, b2a, W2b, b2b, Wf1, bf1, Wf2, bf2):
    n, din = x.shape
    e = edge_index.shape[1]
    dh = W1a.shape[1]
    src = edge_index[0]
    dst = edge_index[1]
    p1 = din // _LANES
    p2 = dh // _LANES
    n_pad = -(-n // (_NS * _ZC)) * (_NS * _ZC)

    ep1 = _edge_proj(edge_feats, We1, be1, p1)
    aggr1 = _segment_messages(
        x.reshape(n * p1, _LANES), ep1.reshape(p1 * e, _LANES),
        src, dst, n_pad, e, p1).reshape(p1, n_pad, _LANES)
    h = _node_mlp1(x, aggr1, W1a, b1a, W1b, b1b)

    ep2 = _edge_proj(edge_feats, We2, be2, p2)
    aggr2 = _segment_messages(
        h.reshape(n * p2, _LANES), ep2.reshape(p2 * e, _LANES),
        src, dst, n_pad, e, p2).reshape(p2, n_pad, _LANES)
    out = _node_mlp2_head(h, aggr2, W2a, b2a, W2b, b2b, Wf1, bf1, Wf2, bf2)
    return out


# speedup vs baseline: 1.0067x; 1.0044x over previous
import functools

import jax
import jax.numpy as jnp
from jax import lax
from jax.experimental import pallas as pl
from jax.experimental.pallas import tpu as pltpu
from jax.experimental.pallas import tpu_sc as plsc

_LANES = 128
_NC = 2
_NS = 16
_KB = 80
_ZC = 40


def _segment_messages(h2, ep2, src, dst, n_pad, e, p):
    ppc = p // _NC
    ept = e // _NS
    nb = ept // _KB
    assert nb % 2 == 1, "pipeline assumes an odd batch count"
    npt = n_pad // _NS
    nz = npt // _ZC

    mesh = plsc.VectorSubcoreMesh(
        core_axis_name="c", subcore_axis_name="s",
        num_cores=_NC, num_subcores=_NS)

    @functools.partial(
        pl.kernel,
        out_type=jax.ShapeDtypeStruct((p * n_pad, _LANES), jnp.float32),
        mesh=mesh,
        scratch_types=[
            pltpu.VMEM((_KB,), jnp.int32),
            pltpu.VMEM((_KB,), jnp.int32),
            pltpu.VMEM((_KB,), jnp.int32),
            pltpu.VMEM((_KB,), jnp.int32),
            pltpu.VMEM((_KB,), jnp.int32),
            pltpu.VMEM((_KB,), jnp.int32),
            pltpu.VMEM((_KB, _LANES), jnp.float32),
            pltpu.VMEM((_KB, _LANES), jnp.float32),
            pltpu.VMEM((_KB, _LANES), jnp.float32),
            pltpu.VMEM((_KB, _LANES), jnp.float32),
            pltpu.VMEM((_ZC, _LANES), jnp.float32),
            pltpu.VMEM_SHARED((n_pad, _LANES), jnp.float32),
            pltpu.SemaphoreType.DMA,
            pltpu.SemaphoreType.DMA,
            pltpu.SemaphoreType.DMA,
            pltpu.SemaphoreType.DMA,
        ],
    )
    def aggr_kernel(h_hbm, ep_hbm, src_hbm, dst_hbm, out_hbm,
                    src0, src1, gid0, gid1, dst0, dst1,
                    rows0, rows1, epv0, epv1, zb_v, acc,
                    si0, si1, sg0, sg1):
        c = lax.axis_index("c")
        s = lax.axis_index("s")

        def zero_zb(r, _):
            for j in range(_LANES // 16):
                zb_v[r, pl.ds(j * 16, 16)] = jnp.zeros((16,), jnp.float32)
            return 0

        lax.fori_loop(0, _ZC, zero_zb, 0)

        def idx_start(b, sv, dv, sem):
            base = s * ept + b * _KB
            pltpu.async_copy(src_hbm.at[pl.ds(base, _KB)], sv, sem)
            pltpu.async_copy(dst_hbm.at[pl.ds(base, _KB)], dv, sem)

        def idx_wait(sv, dv, sem):
            pltpu.make_async_copy(src_hbm.at[pl.ds(0, _KB)], sv, sem).wait()
            pltpu.make_async_copy(dst_hbm.at[pl.ds(0, _KB)], dv, sem).wait()

        def gid_compute(sv, gv, q):
            def gid(i, _):
                sl = pl.ds(i * 16, 16)
                gv[sl] = sv[sl] * p + q
                return 0
            lax.fori_loop(0, _KB // 16, gid, 0)

        def gather_start(b, gv, rv, ev, sem, q):
            base = s * ept + b * _KB
            pltpu.async_copy(h_hbm.at[gv], rv, sem)
            pltpu.async_copy(ep_hbm.at[pl.ds(q * e + base, _KB)], ev, sem)

        def gather_wait(gv, rv, ev, sem):
            pltpu.make_async_copy(h_hbm.at[gv], rv, sem).wait()
            pltpu.make_async_copy(ep_hbm.at[pl.ds(0, _KB)], ev, sem).wait()

        def compute_scatter(rv, ev, dv):
            def relu_row(r, _):
                for j in range(_LANES // 16):
                    sl = pl.ds(j * 16, 16)
                    rv[r, sl] = jnp.maximum(rv[r, sl] + ev[r, sl], 0.0)
                return 0
            lax.fori_loop(0, _KB, relu_row, 0)
            pltpu.sync_copy(rv, acc.at[dv], add=True)

        for ql in range(ppc):
            q = c * ppc + ql

            def zero_acc(z, _):
                pltpu.sync_copy(zb_v, acc.at[pl.ds(s * npt + z * _ZC, _ZC)])
                return 0

            lax.fori_loop(0, nz, zero_acc, 0)
            plsc.subcore_barrier()

            idx_start(0, src0, dst0, si0)
            idx_wait(src0, dst0, si0)
            gid_compute(src0, gid0, q)
            gather_start(0, gid0, rows0, epv0, sg0, q)
            idx_start(1, src1, dst1, si1)

            def pair(t, _):
                b = 2 * t
                idx_wait(src1, dst1, si1)
                gid_compute(src1, gid1, q)
                gather_start(b + 1, gid1, rows1, epv1, sg1, q)
                gather_wait(gid0, rows0, epv0, sg0)
                compute_scatter(rows0, epv0, dst0)
                idx_start(b + 2, src0, dst0, si0)
                idx_wait(src0, dst0, si0)
                gid_compute(src0, gid0, q)
                gather_start(b + 2, gid0, rows0, epv0, sg0, q)
                gather_wait(gid1, rows1, epv1, sg1)
                compute_scatter(rows1, epv1, dst1)

                @pl.when(b + 3 < nb)
                def _():
                    idx_start(b + 3, src1, dst1, si1)
                return 0

            lax.fori_loop(0, (nb - 1) // 2, pair, 0)
            gather_wait(gid0, rows0, epv0, sg0)
            compute_scatter(rows0, epv0, dst0)

            plsc.subcore_barrier()

            off = s * npt
            pltpu.sync_copy(
                acc.at[pl.ds(off, npt)],
                out_hbm.at[pl.ds(q * n_pad + off, npt)])

    return aggr_kernel(h2, ep2, src, dst)


def _edge_proj(ef, W, b, p):
    e, de = ef.shape
    dout = W.shape[1]
    be_blk = 4000

    def kern(ef_ref, w_ref, b_ref, out_ref):
        acc = jnp.dot(ef_ref[...], w_ref[...],
                      preferred_element_type=jnp.float32) + b_ref[...]
        for qq in range(p):
            out_ref[qq] = acc[:, qq * _LANES:(qq + 1) * _LANES]

    return pl.pallas_call(
        kern,
        grid=(e // be_blk,),
        in_specs=[
            pl.BlockSpec((be_blk, de), lambda i: (i, 0)),
            pl.BlockSpec((de, dout), lambda i: (0, 0)),
            pl.BlockSpec((1, dout), lambda i: (0, 0)),
        ],
        out_specs=pl.BlockSpec((p, be_blk, _LANES), lambda i: (0, i, 0)),
        out_shape=jax.ShapeDtypeStruct((p, e, _LANES), jnp.float32),
    )(ef, W, b.reshape(1, dout))


def _node_mlp1(x, aggr, Wa, ba, Wb, bb):
    n, din = x.shape
    p = aggr.shape[0]
    dh = Wa.shape[1]
    bn = 2000

    def kern(x_ref, a_ref, wa_ref, ba_ref, wb_ref, bb_ref, out_ref):
        g = x_ref[...] + jnp.concatenate(
            [a_ref[qq] for qq in range(p)], axis=1)
        t = jnp.maximum(
            jnp.dot(g, wa_ref[...], preferred_element_type=jnp.float32)
            + ba_ref[...], 0.0)
        out_ref[...] = jnp.tanh(
            jnp.dot(t, wb_ref[...], preferred_element_type=jnp.float32)
            + bb_ref[...])

    return pl.pallas_call(
        kern,
        grid=(n // bn,),
        in_specs=[
            pl.BlockSpec((bn, din), lambda i: (i, 0)),
            pl.BlockSpec((p, bn, _LANES), lambda i: (0, i, 0)),
            pl.BlockSpec((din, dh), lambda i: (0, 0)),
            pl.BlockSpec((1, dh), lambda i: (0, 0)),
            pl.BlockSpec((dh, dh), lambda i: (0, 0)),
            pl.BlockSpec((1, dh), lambda i: (0, 0)),
        ],
        out_specs=pl.BlockSpec((bn, dh), lambda i: (i, 0)),
        out_shape=jax.ShapeDtypeStruct((n, dh), jnp.float32),
    )(x, aggr, Wa, ba.reshape(1, dh), Wb, bb.reshape(1, dh))


def _node_mlp2_head(h, aggr, Wa, ba, Wb, bb, Wf1, bf1, Wf2, bf2):
    n, dh = h.shape
    p = aggr.shape[0]
    dout = Wf2.shape[1]
    bn = 2000

    def kern(h_ref, a_ref, wa_ref, ba_ref, wb_ref, bb_ref,
             wf1_ref, bf1_ref, wf2_ref, bf2_ref, out_ref):
        g = h_ref[...] + jnp.concatenate(
            [a_ref[qq] for qq in range(p)], axis=1)
        t = jnp.maximum(
            jnp.dot(g, wa_ref[...], preferred_element_type=jnp.float32)
            + ba_ref[...], 0.0)
        t = jnp.tanh(
            jnp.dot(t, wb_ref[...], preferred_element_type=jnp.float32)
            + bb_ref[...])
        t = jnp.tanh(
            jnp.dot(t, wf1_ref[...], preferred_element_type=jnp.float32)
            + bf1_ref[...])
        out_ref[...] = (
            jnp.dot(t, wf2_ref[...], preferred_element_type=jnp.float32)
            + bf2_ref[...])

    return pl.pallas_call(
        kern,
        grid=(n // bn,),
        in_specs=[
            pl.BlockSpec((bn, dh), lambda i: (i, 0)),
            pl.BlockSpec((p, bn, _LANES), lambda i: (0, i, 0)),
            pl.BlockSpec((dh, dh), lambda i: (0, 0)),
            pl.BlockSpec((1, dh), lambda i: (0, 0)),
            pl.BlockSpec((dh, dh), lambda i: (0, 0)),
            pl.BlockSpec((1, dh), lambda i: (0, 0)),
            pl.BlockSpec((dh, dh), lambda i: (0, 0)),
            pl.BlockSpec((1, dh), lambda i: (0, 0)),
            pl.BlockSpec((dh, dout), lambda i: (0, 0)),
            pl.BlockSpec((1, dout), lambda i: (0, 0)),
        ],
        out_specs=pl.BlockSpec((bn, dout), lambda i: (i, 0)),
        out_shape=jax.ShapeDtypeStruct((n, dout), jnp.float32),
    )(h, aggr, Wa, ba.reshape(1, dh), Wb, bb.reshape(1, dh),
      Wf1, bf1.reshape(1, dh), Wf2, bf2.reshape(1, dout))


def kernel(x, edge_index, edge_feats, We1, be1, W1a, b1a, W1b, b1b,
           We2, be2, W2a, b2a, W2b, b2b, Wf1, bf1, Wf2, bf2):
    n, din = x.shape
    e = edge_index.shape[1]
    dh = W1a.shape[1]
    src = edge_index[0]
    dst = edge_index[1]
    p1 = din // _LANES
    p2 = dh // _LANES
    n_pad = -(-n // (_NS * _ZC)) * (_NS * _ZC)

    ep1 = _edge_proj(edge_feats, We1, be1, p1)
    aggr1 = _segment_messages(
        x.reshape(n * p1, _LANES), ep1.reshape(p1 * e, _LANES),
        src, dst, n_pad, e, p1).reshape(p1, n_pad, _LANES)
    h = _node_mlp1(x, aggr1, W1a, b1a, W1b, b1b)

    ep2 = _edge_proj(edge_feats, We2, be2, p2)
    aggr2 = _segment_messages(
        h.reshape(n * p2, _LANES), ep2.reshape(p2 * e, _LANES),
        src, dst, n_pad, e, p2).reshape(p2, n_pad, _LANES)
    out = _node_mlp2_head(h, aggr2, W2a, b2a, W2b, b2b, Wf1, bf1, Wf2, bf2)
    return out


# speedup vs baseline: 1.0907x; 1.0834x over previous
import functools

import jax
import jax.numpy as jnp
from jax import lax
from jax.experimental import pallas as pl
from jax.experimental.pallas import tpu as pltpu
from jax.experimental.pallas import tpu_sc as plsc

_LANES = 128
_NC = 2
_NS = 16
_KB = 80
_ZC = 40


def _segment_messages(h2, ep2, src, dst, n_pad, e, p):
    ppc = p // _NC
    ept = e // _NS
    nb = ept // _KB
    assert nb % 2 == 1, "pipeline assumes an odd batch count"
    npt = n_pad // _NS
    nz = npt // _ZC

    mesh = plsc.VectorSubcoreMesh(
        core_axis_name="c", subcore_axis_name="s",
        num_cores=_NC, num_subcores=_NS)

    @functools.partial(
        pl.kernel,
        out_type=jax.ShapeDtypeStruct((p * n_pad, _LANES), jnp.float32),
        mesh=mesh,
        scratch_types=[
            pltpu.VMEM((_KB,), jnp.int32),
            pltpu.VMEM((_KB,), jnp.int32),
            pltpu.VMEM((_KB,), jnp.int32),
            pltpu.VMEM((_KB,), jnp.int32),
            pltpu.VMEM((_KB,), jnp.int32),
            pltpu.VMEM((_KB,), jnp.int32),
            pltpu.VMEM((_KB, _LANES), jnp.float32),
            pltpu.VMEM((_KB, _LANES), jnp.float32),
            pltpu.VMEM((_KB, _LANES), jnp.float32),
            pltpu.VMEM((_KB, _LANES), jnp.float32),
            pltpu.VMEM((_KB,), jnp.int32),
            pltpu.VMEM((_KB,), jnp.int32),
            pltpu.VMEM((_ZC, _LANES), jnp.float32),
            pltpu.VMEM_SHARED((n_pad, _LANES), jnp.float32),
            pltpu.SemaphoreType.DMA,
            pltpu.SemaphoreType.DMA,
            pltpu.SemaphoreType.DMA,
            pltpu.SemaphoreType.DMA,
            pltpu.SemaphoreType.DMA,
            pltpu.SemaphoreType.DMA,
        ],
    )
    def aggr_kernel(h_hbm, ep_hbm, src_hbm, dst_hbm, out_hbm,
                    src0, src1, gid0, gid1, dst0, dst1,
                    rows0, rows1, epv0, epv1, sdst0, sdst1, zb_v, acc,
                    si0, si1, sg0, sg1, sc0, sc1):
        c = lax.axis_index("c")
        s = lax.axis_index("s")

        def zero_zb(r, _):
            for j in range(_LANES // 16):
                zb_v[r, pl.ds(j * 16, 16)] = jnp.zeros((16,), jnp.float32)
            return 0

        lax.fori_loop(0, _ZC, zero_zb, 0)

        def idx_start(b, sv, dv, sem):
            base = s * ept + b * _KB
            pltpu.async_copy(src_hbm.at[pl.ds(base, _KB)], sv, sem)
            pltpu.async_copy(dst_hbm.at[pl.ds(base, _KB)], dv, sem)

        def idx_wait(sv, dv, sem):
            pltpu.make_async_copy(src_hbm.at[pl.ds(0, _KB)], sv, sem).wait()
            pltpu.make_async_copy(dst_hbm.at[pl.ds(0, _KB)], dv, sem).wait()

        def gid_compute(sv, gv, q):
            def gid(i, _):
                sl = pl.ds(i * 16, 16)
                gv[sl] = sv[sl] * p + q
                return 0
            lax.fori_loop(0, _KB // 16, gid, 0)

        def gather_start(b, gv, rv, ev, sem, q):
            base = s * ept + b * _KB
            pltpu.async_copy(h_hbm.at[gv], rv, sem)
            pltpu.async_copy(ep_hbm.at[pl.ds(q * e + base, _KB)], ev, sem)

        def gather_wait(gv, rv, ev, sem):
            pltpu.make_async_copy(h_hbm.at[gv], rv, sem).wait()
            pltpu.make_async_copy(ep_hbm.at[pl.ds(0, _KB)], ev, sem).wait()

        def relu_batch(rv, ev):
            def relu_row(r, _):
                for j in range(_LANES // 16):
                    sl = pl.ds(j * 16, 16)
                    rv[r, sl] = jnp.maximum(rv[r, sl] + ev[r, sl], 0.0)
                return 0
            lax.fori_loop(0, _KB, relu_row, 0)

        def compute_scatter(rv, ev, dv):
            relu_batch(rv, ev)
            pltpu.sync_copy(rv, acc.at[dv], add=True)

        def compute_scatter_async(rv, ev, dv, sdv, scm):
            relu_batch(rv, ev)

            def cpd(i, _):
                sl = pl.ds(i * 16, 16)
                sdv[sl] = dv[sl]
                return 0
            lax.fori_loop(0, _KB // 16, cpd, 0)
            pltpu.async_copy(rv, acc.at[sdv], scm, add=True)

        def scatter_wait(rv, sdv, scm):
            pltpu.make_async_copy(rv, acc.at[sdv], scm).wait()

        for ql in range(ppc):
            q = c * ppc + ql

            def zero_acc(z, _):
                pltpu.sync_copy(zb_v, acc.at[pl.ds(s * npt + z * _ZC, _ZC)])
                return 0

            lax.fori_loop(0, nz, zero_acc, 0)
            plsc.subcore_barrier()

            idx_start(0, src0, dst0, si0)
            idx_wait(src0, dst0, si0)
            gid_compute(src0, gid0, q)
            gather_start(0, gid0, rows0, epv0, sg0, q)
            idx_start(1, src1, dst1, si1)

            def pair(t, _):
                b = 2 * t
                idx_wait(src1, dst1, si1)
                gid_compute(src1, gid1, q)

                @pl.when(t > 0)
                def _():
                    scatter_wait(rows1, sdst1, sc1)
                gather_start(b + 1, gid1, rows1, epv1, sg1, q)
                gather_wait(gid0, rows0, epv0, sg0)
                compute_scatter_async(rows0, epv0, dst0, sdst0, sc0)
                idx_start(b + 2, src0, dst0, si0)
                idx_wait(src0, dst0, si0)
                gid_compute(src0, gid0, q)
                scatter_wait(rows0, sdst0, sc0)
                gather_start(b + 2, gid0, rows0, epv0, sg0, q)
                gather_wait(gid1, rows1, epv1, sg1)
                compute_scatter_async(rows1, epv1, dst1, sdst1, sc1)

                @pl.when(b + 3 < nb)
                def _():
                    idx_start(b + 3, src1, dst1, si1)
                return 0

            lax.fori_loop(0, (nb - 1) // 2, pair, 0)
            scatter_wait(rows1, sdst1, sc1)
            gather_wait(gid0, rows0, epv0, sg0)
            compute_scatter(rows0, epv0, dst0)

            plsc.subcore_barrier()

            off = s * npt
            pltpu.sync_copy(
                acc.at[pl.ds(off, npt)],
                out_hbm.at[pl.ds(q * n_pad + off, npt)])

    return aggr_kernel(h2, ep2, src, dst)


def _edge_proj(ef, W, b, p):
    e, de = ef.shape
    dout = W.shape[1]
    be_blk = 4000

    def kern(ef_ref, w_ref, b_ref, out_ref):
        acc = jnp.dot(ef_ref[...], w_ref[...],
                      preferred_element_type=jnp.float32) + b_ref[...]
        for qq in range(p):
            out_ref[qq] = acc[:, qq * _LANES:(qq + 1) * _LANES]

    return pl.pallas_call(
        kern,
        grid=(e // be_blk,),
        in_specs=[
            pl.BlockSpec((be_blk, de), lambda i: (i, 0)),
            pl.BlockSpec((de, dout), lambda i: (0, 0)),
            pl.BlockSpec((1, dout), lambda i: (0, 0)),
        ],
        out_specs=pl.BlockSpec((p, be_blk, _LANES), lambda i: (0, i, 0)),
        out_shape=jax.ShapeDtypeStruct((p, e, _LANES), jnp.float32),
    )(ef, W, b.reshape(1, dout))


def _node_mlp1(x, aggr, Wa, ba, Wb, bb):
    n, din = x.shape
    p = aggr.shape[0]
    dh = Wa.shape[1]
    bn = 2000

    def kern(x_ref, a_ref, wa_ref, ba_ref, wb_ref, bb_ref, out_ref):
        g = x_ref[...] + jnp.concatenate(
            [a_ref[qq] for qq in range(p)], axis=1)
        t = jnp.maximum(
            jnp.dot(g, wa_ref[...], preferred_element_type=jnp.float32)
            + ba_ref[...], 0.0)
        out_ref[...] = jnp.tanh(
            jnp.dot(t, wb_ref[...], preferred_element_type=jnp.float32)
            + bb_ref[...])

    return pl.pallas_call(
        kern,
        grid=(n // bn,),
        in_specs=[
            pl.BlockSpec((bn, din), lambda i: (i, 0)),
            pl.BlockSpec((p, bn, _LANES), lambda i: (0, i, 0)),
            pl.BlockSpec((din, dh), lambda i: (0, 0)),
            pl.BlockSpec((1, dh), lambda i: (0, 0)),
            pl.BlockSpec((dh, dh), lambda i: (0, 0)),
            pl.BlockSpec((1, dh), lambda i: (0, 0)),
        ],
        out_specs=pl.BlockSpec((bn, dh), lambda i: (i, 0)),
        out_shape=jax.ShapeDtypeStruct((n, dh), jnp.float32),
    )(x, aggr, Wa, ba.reshape(1, dh), Wb, bb.reshape(1, dh))


def _node_mlp2_head(h, aggr, Wa, ba, Wb, bb, Wf1, bf1, Wf2, bf2):
    n, dh = h.shape
    p = aggr.shape[0]
    dout = Wf2.shape[1]
    bn = 2000

    def kern(h_ref, a_ref, wa_ref, ba_ref, wb_ref, bb_ref,
             wf1_ref, bf1_ref, wf2_ref, bf2_ref, out_ref):
        g = h_ref[...] + jnp.concatenate(
            [a_ref[qq] for qq in range(p)], axis=1)
        t = jnp.maximum(
            jnp.dot(g, wa_ref[...], preferred_element_type=jnp.float32)
            + ba_ref[...], 0.0)
        t = jnp.tanh(
            jnp.dot(t, wb_ref[...], preferred_element_type=jnp.float32)
            + bb_ref[...])
        t = jnp.tanh(
            jnp.dot(t, wf1_ref[...], preferred_element_type=jnp.float32)
            + bf1_ref[...])
        out_ref[...] = (
            jnp.dot(t, wf2_ref[...], preferred_element_type=jnp.float32)
            + bf2_ref[...])

    return pl.pallas_call(
        kern,
        grid=(n // bn,),
        in_specs=[
            pl.BlockSpec((bn, dh), lambda i: (i, 0)),
            pl.BlockSpec((p, bn, _LANES), lambda i: (0, i, 0)),
            pl.BlockSpec((dh, dh), lambda i: (0, 0)),
            pl.BlockSpec((1, dh), lambda i: (0, 0)),
            pl.BlockSpec((dh, dh), lambda i: (0, 0)),
            pl.BlockSpec((1, dh), lambda i: (0, 0)),
            pl.BlockSpec((dh, dh), lambda i: (0, 0)),
            pl.BlockSpec((1, dh), lambda i: (0, 0)),
            pl.BlockSpec((dh, dout), lambda i: (0, 0)),
            pl.BlockSpec((1, dout), lambda i: (0, 0)),
        ],
        out_specs=pl.BlockSpec((bn, dout), lambda i: (i, 0)),
        out_shape=jax.ShapeDtypeStruct((n, dout), jnp.float32),
    )(h, aggr, Wa, ba.reshape(1, dh), Wb, bb.reshape(1, dh),
      Wf1, bf1.reshape(1, dh), Wf2, bf2.reshape(1, dout))


def kernel(x, edge_index, edge_feats, We1, be1, W1a, b1a, W1b, b1b,
           We2, be2, W2a, b2a, W2b, b2b, Wf1, bf1, Wf2, bf2):
    n, din = x.shape
    e = edge_index.shape[1]
    dh = W1a.shape[1]
    src = edge_index[0]
    dst = edge_index[1]
    p1 = din // _LANES
    p2 = dh // _LANES
    n_pad = -(-n // (_NS * _ZC)) * (_NS * _ZC)

    ep1 = _edge_proj(edge_feats, We1, be1, p1)
    aggr1 = _segment_messages(
        x.reshape(n * p1, _LANES), ep1.reshape(p1 * e, _LANES),
        src, dst, n_pad, e, p1).reshape(p1, n_pad, _LANES)
    h = _node_mlp1(x, aggr1, W1a, b1a, W1b, b1b)

    ep2 = _edge_proj(edge_feats, We2, be2, p2)
    aggr2 = _segment_messages(
        h.reshape(n * p2, _LANES), ep2.reshape(p2 * e, _LANES),
        src, dst, n_pad, e, p2).reshape(p2, n_pad, _LANES)
    out = _node_mlp2_head(h, aggr2, W2a, b2a, W2b, b2b, Wf1, bf1, Wf2, bf2)
    return out


# speedup vs baseline: 1.0913x; 1.0006x over previous
import functools

import jax
import jax.numpy as jnp
from jax import lax
from jax.experimental import pallas as pl
from jax.experimental.pallas import tpu as pltpu
from jax.experimental.pallas import tpu_sc as plsc

_LANES = 128
_NC = 2
_NS = 16
_KB = 80
_ZC = 40


def _segment_messages(h2, ep2, src, dst, n_pad, e, p):
    ppc = p // _NC
    ept = e // _NS
    nb = ept // _KB
    assert nb % 2 == 1, "pipeline assumes an odd batch count"
    npt = n_pad // _NS
    nz = npt // _ZC

    mesh = plsc.VectorSubcoreMesh(
        core_axis_name="c", subcore_axis_name="s",
        num_cores=_NC, num_subcores=_NS)

    @functools.partial(
        pl.kernel,
        out_type=jax.ShapeDtypeStruct((p * n_pad, _LANES), jnp.float32),
        mesh=mesh,
        scratch_types=[
            pltpu.VMEM((_KB,), jnp.int32),
            pltpu.VMEM((_KB,), jnp.int32),
            pltpu.VMEM((_KB,), jnp.int32),
            pltpu.VMEM((_KB,), jnp.int32),
            pltpu.VMEM((_KB,), jnp.int32),
            pltpu.VMEM((_KB,), jnp.int32),
            pltpu.VMEM((_KB, _LANES), jnp.float32),
            pltpu.VMEM((_KB, _LANES), jnp.float32),
            pltpu.VMEM((_KB, _LANES), jnp.float32),
            pltpu.VMEM((_KB, _LANES), jnp.float32),
            pltpu.VMEM((_KB,), jnp.int32),
            pltpu.VMEM((_KB,), jnp.int32),
            pltpu.VMEM((_ZC, _LANES), jnp.float32),
            pltpu.VMEM_SHARED((n_pad, _LANES), jnp.float32),
            pltpu.SemaphoreType.DMA,
            pltpu.SemaphoreType.DMA,
            pltpu.SemaphoreType.DMA,
            pltpu.SemaphoreType.DMA,
            pltpu.SemaphoreType.DMA,
            pltpu.SemaphoreType.DMA,
        ],
    )
    def aggr_kernel(h_hbm, ep_hbm, src_hbm, dst_hbm, out_hbm,
                    src0, src1, gid0, gid1, dst0, dst1,
                    rows0, rows1, epv0, epv1, sdst0, sdst1, zb_v, acc,
                    si0, si1, sg0, sg1, sc0, sc1):
        c = lax.axis_index("c")
        s = lax.axis_index("s")

        def zero_zb(r, _):
            for j in range(_LANES // 16):
                zb_v[r, pl.ds(j * 16, 16)] = jnp.zeros((16,), jnp.float32)
            return 0

        lax.fori_loop(0, _ZC, zero_zb, 0)

        def idx_start(b, sv, dv, sem):
            base = s * ept + b * _KB
            pltpu.async_copy(src_hbm.at[pl.ds(base, _KB)], sv, sem)
            pltpu.async_copy(dst_hbm.at[pl.ds(base, _KB)], dv, sem)

        def idx_wait(sv, dv, sem):
            pltpu.make_async_copy(src_hbm.at[pl.ds(0, _KB)], sv, sem).wait()
            pltpu.make_async_copy(dst_hbm.at[pl.ds(0, _KB)], dv, sem).wait()

        def gid_compute(sv, gv, q):
            def gid(i, _):
                sl = pl.ds(i * 16, 16)
                gv[sl] = sv[sl] * p + q
                return 0
            lax.fori_loop(0, _KB // 16, gid, 0)

        def gather_start(b, gv, rv, ev, sem, q):
            base = s * ept + b * _KB
            pltpu.async_copy(h_hbm.at[gv], rv, sem)
            pltpu.async_copy(ep_hbm.at[pl.ds(q * e + base, _KB)], ev, sem)

        def gather_wait(gv, rv, ev, sem):
            pltpu.make_async_copy(h_hbm.at[gv], rv, sem).wait()
            pltpu.make_async_copy(ep_hbm.at[pl.ds(0, _KB)], ev, sem).wait()

        def relu_batch(rv, ev):
            def relu_row(t, _):
                for u in range(2):
                    r = 2 * t + u
                    for j in range(_LANES // 16):
                        sl = pl.ds(j * 16, 16)
                        rv[r, sl] = jnp.maximum(rv[r, sl] + ev[r, sl], 0.0)
                return 0
            lax.fori_loop(0, _KB // 2, relu_row, 0)

        def compute_scatter(rv, ev, dv):
            relu_batch(rv, ev)
            pltpu.sync_copy(rv, acc.at[dv], add=True)

        def compute_scatter_async(rv, ev, dv, sdv, scm):
            relu_batch(rv, ev)

            def cpd(i, _):
                sl = pl.ds(i * 16, 16)
                sdv[sl] = dv[sl]
                return 0
            lax.fori_loop(0, _KB // 16, cpd, 0)
            pltpu.async_copy(rv, acc.at[sdv], scm, add=True)

        def scatter_wait(rv, sdv, scm):
            pltpu.make_async_copy(rv, acc.at[sdv], scm).wait()

        for ql in range(ppc):
            q = c * ppc + ql

            def zero_acc(z, _):
                pltpu.sync_copy(zb_v, acc.at[pl.ds(s * npt + z * _ZC, _ZC)])
                return 0

            lax.fori_loop(0, nz, zero_acc, 0)
            plsc.subcore_barrier()

            idx_start(0, src0, dst0, si0)
            idx_wait(src0, dst0, si0)
            gid_compute(src0, gid0, q)
            gather_start(0, gid0, rows0, epv0, sg0, q)
            idx_start(1, src1, dst1, si1)

            def pair(t, _):
                b = 2 * t
                idx_wait(src1, dst1, si1)
                gid_compute(src1, gid1, q)

                @pl.when(t > 0)
                def _():
                    scatter_wait(rows1, sdst1, sc1)
                gather_start(b + 1, gid1, rows1, epv1, sg1, q)
                gather_wait(gid0, rows0, epv0, sg0)
                compute_scatter_async(rows0, epv0, dst0, sdst0, sc0)
                idx_start(b + 2, src0, dst0, si0)
                idx_wait(src0, dst0, si0)
                gid_compute(src0, gid0, q)
                scatter_wait(rows0, sdst0, sc0)
                gather_start(b + 2, gid0, rows0, epv0, sg0, q)
                gather_wait(gid1, rows1, epv1, sg1)
                compute_scatter_async(rows1, epv1, dst1, sdst1, sc1)

                @pl.when(b + 3 < nb)
                def _():
                    idx_start(b + 3, src1, dst1, si1)
                return 0

            lax.fori_loop(0, (nb - 1) // 2, pair, 0)
            scatter_wait(rows1, sdst1, sc1)
            gather_wait(gid0, rows0, epv0, sg0)
            compute_scatter(rows0, epv0, dst0)

            plsc.subcore_barrier()

            off = s * npt
            pltpu.sync_copy(
                acc.at[pl.ds(off, npt)],
                out_hbm.at[pl.ds(q * n_pad + off, npt)])

    return aggr_kernel(h2, ep2, src, dst)


def _edge_proj(ef, W, b, p):
    e, de = ef.shape
    dout = W.shape[1]
    be_blk = 4000

    def kern(ef_ref, w_ref, b_ref, out_ref):
        acc = jnp.dot(ef_ref[...], w_ref[...],
                      preferred_element_type=jnp.float32) + b_ref[...]
        for qq in range(p):
            out_ref[qq] = acc[:, qq * _LANES:(qq + 1) * _LANES]

    return pl.pallas_call(
        kern,
        grid=(e // be_blk,),
        in_specs=[
            pl.BlockSpec((be_blk, de), lambda i: (i, 0)),
            pl.BlockSpec((de, dout), lambda i: (0, 0)),
            pl.BlockSpec((1, dout), lambda i: (0, 0)),
        ],
        out_specs=pl.BlockSpec((p, be_blk, _LANES), lambda i: (0, i, 0)),
        out_shape=jax.ShapeDtypeStruct((p, e, _LANES), jnp.float32),
    )(ef, W, b.reshape(1, dout))


def _node_mlp1(x, aggr, Wa, ba, Wb, bb):
    n, din = x.shape
    p = aggr.shape[0]
    dh = Wa.shape[1]
    bn = 2000

    def kern(x_ref, a_ref, wa_ref, ba_ref, wb_ref, bb_ref, out_ref):
        g = x_ref[...] + jnp.concatenate(
            [a_ref[qq] for qq in range(p)], axis=1)
        t = jnp.maximum(
            jnp.dot(g, wa_ref[...], preferred_element_type=jnp.float32)
            + ba_ref[...], 0.0)
        out_ref[...] = jnp.tanh(
            jnp.dot(t, wb_ref[...], preferred_element_type=jnp.float32)
            + bb_ref[...])

    return pl.pallas_call(
        kern,
        grid=(n // bn,),
        in_specs=[
            pl.BlockSpec((bn, din), lambda i: (i, 0)),
            pl.BlockSpec((p, bn, _LANES), lambda i: (0, i, 0)),
            pl.BlockSpec((din, dh), lambda i: (0, 0)),
            pl.BlockSpec((1, dh), lambda i: (0, 0)),
            pl.BlockSpec((dh, dh), lambda i: (0, 0)),
            pl.BlockSpec((1, dh), lambda i: (0, 0)),
        ],
        out_specs=pl.BlockSpec((bn, dh), lambda i: (i, 0)),
        out_shape=jax.ShapeDtypeStruct((n, dh), jnp.float32),
    )(x, aggr, Wa, ba.reshape(1, dh), Wb, bb.reshape(1, dh))


def _node_mlp2_head(h, aggr, Wa, ba, Wb, bb, Wf1, bf1, Wf2, bf2):
    n, dh = h.shape
    p = aggr.shape[0]
    dout = Wf2.shape[1]
    bn = 2000

    def kern(h_ref, a_ref, wa_ref, ba_ref, wb_ref, bb_ref,
             wf1_ref, bf1_ref, wf2_ref, bf2_ref, out_ref):
        g = h_ref[...] + jnp.concatenate(
            [a_ref[qq] for qq in range(p)], axis=1)
        t = jnp.maximum(
            jnp.dot(g, wa_ref[...], preferred_element_type=jnp.float32)
            + ba_ref[...], 0.0)
        t = jnp.tanh(
            jnp.dot(t, wb_ref[...], preferred_element_type=jnp.float32)
            + bb_ref[...])
        t = jnp.tanh(
            jnp.dot(t, wf1_ref[...], preferred_element_type=jnp.float32)
            + bf1_ref[...])
        out_ref[...] = (
            jnp.dot(t, wf2_ref[...], preferred_element_type=jnp.float32)
            + bf2_ref[...])

    return pl.pallas_call(
        kern,
        grid=(n // bn,),
        in_specs=[
            pl.BlockSpec((bn, dh), lambda i: (i, 0)),
            pl.BlockSpec((p, bn, _LANES), lambda i: (0, i, 0)),
            pl.BlockSpec((dh, dh), lambda i: (0, 0)),
            pl.BlockSpec((1, dh), lambda i: (0, 0)),
            pl.BlockSpec((dh, dh), lambda i: (0, 0)),
            pl.BlockSpec((1, dh), lambda i: (0, 0)),
            pl.BlockSpec((dh, dh), lambda i: (0, 0)),
            pl.BlockSpec((1, dh), lambda i: (0, 0)),
            pl.BlockSpec((dh, dout), lambda i: (0, 0)),
            pl.BlockSpec((1, dout), lambda i: (0, 0)),
        ],
        out_specs=pl.BlockSpec((bn, dout), lambda i: (i, 0)),
        out_shape=jax.ShapeDtypeStruct((n, dout), jnp.float32),
    )(h, aggr, Wa, ba.reshape(1, dh), Wb, bb.reshape(1, dh),
      Wf1, bf1.reshape(1, dh), Wf2, bf2.reshape(1, dout))


def kernel(x, edge_index, edge_feats, We1, be1, W1a, b1a, W1b, b1b,
           We2, be2, W2a, b2a, W2b, b2b, Wf1, bf1, Wf2, bf2):
    n, din = x.shape
    e = edge_index.shape[1]
    dh = W1a.shape[1]
    src = edge_index[0]
    dst = edge_index[1]
    p1 = din // _LANES
    p2 = dh // _LANES
    n_pad = -(-n // (_NS * _ZC)) * (_NS * _ZC)

    ep1 = _edge_proj(edge_feats, We1, be1, p1)
    aggr1 = _segment_messages(
        x.reshape(n * p1, _LANES), ep1.reshape(p1 * e, _LANES),
        src, dst, n_pad, e, p1).reshape(p1, n_pad, _LANES)
    h = _node_mlp1(x, aggr1, W1a, b1a, W1b, b1b)

    ep2 = _edge_proj(edge_feats, We2, be2, p2)
    aggr2 = _segment_messages(
        h.reshape(n * p2, _LANES), ep2.reshape(p2 * e, _LANES),
        src, dst, n_pad, e, p2).reshape(p2, n_pad, _LANES)
    out = _node_mlp2_head(h, aggr2, W2a, b2a, W2b, b2b, Wf1, bf1, Wf2, bf2)
    return out
